# Initial kernel scaffold; baseline (speedup 1.0000x reference)
import jax
import jax.numpy as jnp
from jax.experimental import pallas as pl


def kernel(node_features, nodepair, edge_attribute, node_neighbour_index, We1, be1, We2, be2, Wg1, bg1, Wg2, bg2):
    raise NotImplementedError("write your pallas kernel here")


# speedup vs baseline: 1.2413x; 1.2413x over previous
import functools

import jax
import jax.numpy as jnp
from jax import lax
from jax.experimental import pallas as pl
from jax.experimental.pallas import tpu as pltpu
from jax.experimental.pallas import tpu_sc as plsc

_CH = 80
_BE = 512


def _sc_geometry():
    try:
        info = plsc.get_sparse_core_info()
        return info.num_cores, info.num_subcores
    except Exception:
        return 2, 16


def _make_gather(N, D, E, NC, NS):
    NW = NC * NS
    eper = E // NW
    nch = eper // _CH
    mesh = plsc.VectorSubcoreMesh(core_axis_name="c", subcore_axis_name="s")

    @functools.partial(
        pl.kernel,
        mesh=mesh,
        out_type=(
            jax.ShapeDtypeStruct((E, D), jnp.float32),
            jax.ShapeDtypeStruct((E, D), jnp.float32),
        ),
        scratch_types=[
            pltpu.VMEM((_CH,), jnp.int32),
            pltpu.VMEM((_CH,), jnp.int32),
            pltpu.VMEM((_CH, D), jnp.float32),
            pltpu.VMEM((_CH, D), jnp.float32),
            pltpu.SemaphoreType.DMA,
            pltpu.SemaphoreType.DMA,
        ],
    )
    def gather_kernel(x_hbm, src_hbm, dst_hbm, xs_hbm, xd_hbm,
                      sidx, didx, sbuf, dbuf, ssem, dsem):
        wid = lax.axis_index("s") * NC + lax.axis_index("c")
        ebase = wid * eper

        def body(j, carry):
            off = j * _CH
            pltpu.sync_copy(src_hbm.at[pl.ds(ebase + off, _CH)], sidx)
            pltpu.sync_copy(dst_hbm.at[pl.ds(ebase + off, _CH)], didx)
            cs = pltpu.async_copy(x_hbm.at[sidx], sbuf, ssem)
            cd = pltpu.async_copy(x_hbm.at[didx], dbuf, dsem)
            cs.wait()
            pltpu.sync_copy(sbuf, xs_hbm.at[pl.ds(ebase + off, _CH)])
            cd.wait()
            pltpu.sync_copy(dbuf, xd_hbm.at[pl.ds(ebase + off, _CH)])
            return carry

        lax.fori_loop(0, nch, body, 0)

    return gather_kernel


def _edge_mlp_block(xs_ref, xd_ref, ea_ref, We1_r, be1_r, We2_r, be2_r,
                    Wa_r, Wb_r, We_r, bg1_r, Wg2_r, bg2_r, out_ref):
    f32 = jnp.float32
    h1 = jnp.maximum(
        jnp.dot(ea_ref[:], We1_r[:], preferred_element_type=f32) + be1_r[:], 0.0)
    e = jnp.dot(h1, We2_r[:], preferred_element_type=f32) + be2_r[:]
    pre = (jnp.dot(xs_ref[:], Wa_r[:], preferred_element_type=f32)
           + jnp.dot(xd_ref[:], Wb_r[:], preferred_element_type=f32)
           + jnp.dot(e, We_r[:], preferred_element_type=f32)
           + bg1_r[:])
    out_ref[:] = (jnp.dot(jnp.maximum(pre, 0.0), Wg2_r[:],
                          preferred_element_type=f32) + bg2_r[:])


def _edge_new_tc(xs, xd, ea, We1, be1, We2, be2, Wg1, bg1, Wg2, bg2):
    E, D = xs.shape
    DE = ea.shape[1]
    HE = We1.shape[1]
    OE = We2.shape[1]
    HG = Wg1.shape[1]
    OG = Wg2.shape[1]
    BE = 2000
    grid = (E // BE,)
    Wa = Wg1[:D]
    Wb = Wg1[D:2 * D]
    We = Wg1[2 * D:]

    def full(shape):
        return pl.BlockSpec(shape, lambda i: (0,) * len(shape))

    return pl.pallas_call(
        _edge_mlp_block,
        grid=grid,
        in_specs=[
            pl.BlockSpec((BE, D), lambda i: (i, 0)),
            pl.BlockSpec((BE, D), lambda i: (i, 0)),
            pl.BlockSpec((BE, DE), lambda i: (i, 0)),
            full((DE, HE)), full((1, HE)),
            full((HE, OE)), full((1, OE)),
            full((D, HG)), full((D, HG)), full((OE, HG)), full((1, HG)),
            full((HG, OG)), full((1, OG)),
        ],
        out_specs=pl.BlockSpec((BE, OG), lambda i: (i, 0)),
        out_shape=jax.ShapeDtypeStruct((E, OG), jnp.float32),
    )(xs, xd, ea, We1, be1.reshape(1, -1), We2, be2.reshape(1, -1),
      Wa, Wb, We, bg1.reshape(1, -1), Wg2, bg2.reshape(1, -1))


def _hist_block_body(NB):
    def body(nbr_ref, out_ref):
        o = nbr_ref[0, 0] // NB
        iota = lax.broadcasted_iota(jnp.int32, (_BE, 32), 1)
        oh = (o[:, None] == iota).astype(jnp.float32)
        out_ref[0, 0] = jnp.sum(oh, axis=0)
    return body


def _hist_tc(nbr3, NB):
    NBLK = nbr3.shape[0]
    return pl.pallas_call(
        _hist_block_body(NB),
        grid=(NBLK,),
        in_specs=[pl.BlockSpec((1, 1, _BE), lambda i: (i, 0, 0))],
        out_specs=pl.BlockSpec((1, 1, 32), lambda i: (i, 0, 0)),
        out_shape=jax.ShapeDtypeStruct((NBLK, 1, 32), jnp.float32),
    )(nbr3)


def _pos_block_body(NB):
    def body(nbr_ref, bases_ref, tril_ref, out_ref):
        o = nbr_ref[0, 0] // NB
        iota = lax.broadcasted_iota(jnp.int32, (_BE, 32), 1)
        oh = (o[:, None] == iota).astype(jnp.float32)
        rank = jnp.dot(tril_ref[:], oh, preferred_element_type=jnp.float32)
        base = jnp.sum(bases_ref[0] * oh, axis=1)
        p = base + jnp.sum(rank * oh, axis=1)
        out_ref[0, 0] = p.astype(jnp.int32)
    return body


def _pos_tc(nbr3, bases3, tril, NB):
    NBLK = nbr3.shape[0]
    return pl.pallas_call(
        _pos_block_body(NB),
        grid=(NBLK,),
        in_specs=[
            pl.BlockSpec((1, 1, _BE), lambda i: (i, 0, 0)),
            pl.BlockSpec((1, 1, 32), lambda i: (i, 0, 0)),
            pl.BlockSpec((_BE, _BE), lambda i: (0, 0)),
        ],
        out_specs=pl.BlockSpec((1, 1, _BE), lambda i: (i, 0, 0)),
        out_shape=jax.ShapeDtypeStruct((NBLK, 1, _BE), jnp.int32),
    )(nbr3, bases3, tril)


def _make_permute(N, D, E, NC, NS):
    NW = NC * NS
    eper = E // NW
    nch = eper // _CH
    EP = E + _CH
    mesh = plsc.VectorSubcoreMesh(core_axis_name="c", subcore_axis_name="s")

    @functools.partial(
        pl.kernel,
        mesh=mesh,
        out_type=(
            jax.ShapeDtypeStruct((EP, D), jnp.float32),
            jax.ShapeDtypeStruct((EP, 16), jnp.int32),
        ),
        scratch_types=[
            pltpu.VMEM((_CH, D), jnp.float32),
            pltpu.VMEM((_CH,), jnp.int32),
            pltpu.VMEM((_CH,), jnp.int32),
            pltpu.VMEM((16, 16), jnp.int32),
            pltpu.SemaphoreType.DMA,
            pltpu.SemaphoreType.DMA,
        ],
    )
    def permute_kernel(e_hbm, nbr_hbm, p_hbm, srow_hbm, smeta_hbm,
                       ebuf, nbrbuf, pbuf, mstage, rsem, msem):
        wid = lax.axis_index("s") * NC + lax.axis_index("c")
        ebase = wid * eper
        iota16 = lax.iota(jnp.int32, 16)
        l0 = 1 - jnp.minimum(iota16, 1)
        l1 = jnp.maximum(0, 1 - jnp.abs(iota16 - 1))

        def chunk(j, carry):
            off = j * _CH
            pltpu.sync_copy(e_hbm.at[pl.ds(ebase + off, _CH)], ebuf)
            pltpu.sync_copy(nbr_hbm.at[pl.ds(ebase + off, _CH)], nbrbuf)
            pltpu.sync_copy(p_hbm.at[pl.ds(ebase + off, _CH)], pbuf)

            def group(g, carry2):
                pv = pbuf[pl.ds(g * 16, 16)]
                nv = nbrbuf[pl.ds(g * 16, 16)]
                copies = []
                for lane in range(16):
                    k = g * 16 + lane
                    eid = ebase + off + k
                    meta = eid * l0 + nv[lane] * l1
                    mstage[lane, :] = meta
                    pk = pv[lane]
                    copies.append(pltpu.async_copy(
                        ebuf.at[k], srow_hbm.at[pk], rsem))
                    copies.append(pltpu.async_copy(
                        mstage.at[lane], smeta_hbm.at[pk], msem))
                for c in copies:
                    c.wait()
                return carry2

            lax.fori_loop(0, _CH // 16, group, 0)
            return carry

        lax.fori_loop(0, nch, chunk, 0)

    return permute_kernel


def _make_accumulate(N, D, E, NC, NS):
    NW = NC * NS
    NB = (-(-N // NW) + 7) // 8 * 8
    DW = D // 16
    last_rows = N - (NW - 1) * NB
    EP = E + _CH
    mesh = plsc.VectorSubcoreMesh(core_axis_name="c", subcore_axis_name="s")

    @functools.partial(
        pl.kernel,
        mesh=mesh,
        out_type=jax.ShapeDtypeStruct((N * D,), jnp.float32),
        scratch_types=[
            pltpu.VMEM((_CH, D), jnp.float32),
            pltpu.VMEM((_CH, 16), jnp.int32),
            pltpu.VMEM((48,), jnp.int32),
            pltpu.VMEM(((NB + 8) * D,), jnp.float32),
            pltpu.VMEM(((NB + 8) * 16,), jnp.float32),
        ],
    )
    def acc_kernel(srow_hbm, smeta_hbm, segb_hbm, out_hbm,
                   srbuf, smbuf, segbuf, acc, cnt):
        cid = lax.axis_index("c")
        sid = lax.axis_index("s")
        wid = sid * NC + cid
        lo = wid * NB
        zf16 = jnp.zeros((16,), jnp.float32)
        one16 = jnp.ones((16,), jnp.float32)

        def zacc(i, c):
            acc[pl.ds(i * 16, 16)] = zf16
            return c

        lax.fori_loop(0, (NB + 8) * D // 16, zacc, 0)

        def zcnt(i, c):
            cnt[pl.ds(i * 16, 16)] = zf16
            return c

        lax.fori_loop(0, NB + 8, zcnt, 0)

        pltpu.sync_copy(segb_hbm, segbuf)
        sv0 = segbuf[pl.ds(0, 16)]
        sv1 = segbuf[pl.ds(16, 16)]
        sv2 = segbuf[pl.ds(32, 16)]
        b_lo = jnp.int32(0)
        b_hi = jnp.int32(0)
        for t in range(33):
            val = (sv0, sv1, sv2)[t // 16][t % 16]
            b_lo = jnp.where(wid == t, val, b_lo)
            b_hi = jnp.where(wid + 1 == t, val, b_hi)

        start = pl.multiple_of(b_lo & jnp.int32(-8), 8)
        nchunks = (b_hi - start + _CH - 1) // _CH

        def chunk(j, carry):
            g0 = pl.multiple_of(start + j * _CH, 8)
            pltpu.sync_copy(srow_hbm.at[pl.ds(g0, _CH)], srbuf)
            pltpu.sync_copy(smeta_hbm.at[pl.ds(g0, _CH)], smbuf)

            def edge(k, c2):
                mrow = smbuf[k, pl.ds(0, 16)]
                r0 = mrow[1] - lo
                kg = g0 + k
                r1 = jnp.where(kg >= b_hi, NB, r0)
                r1 = jnp.where(r0 < 0, NB, r1)
                r1 = jnp.where(r0 >= NB, NB, r1)
                base = r1 * D
                for j8 in range(DW):
                    acc[pl.ds(base + j8 * 16, 16)] = (
                        acc[pl.ds(base + j8 * 16, 16)]
                        + srbuf[k, pl.ds(j8 * 16, 16)])
                cb = r1 * 16
                cnt[pl.ds(cb, 16)] = cnt[pl.ds(cb, 16)] + one16
                return c2

            lax.fori_loop(0, _CH, edge, 0)
            return carry

        lax.fori_loop(0, nchunks, chunk, 0)

        def div_row(r, c):
            cvec = jnp.maximum(cnt[pl.ds(r * 16, 16)], 1.0)
            for j8 in range(DW):
                acc[pl.ds(r * D + j8 * 16, 16)] = (
                    acc[pl.ds(r * D + j8 * 16, 16)] / cvec)
            return c

        lax.fori_loop(0, NB, div_row, 0)

        @pl.when(wid < NW - 1)
        def _out_full():
            pltpu.sync_copy(acc.at[pl.ds(0, NB * D)],
                            out_hbm.at[pl.ds(lo * D, NB * D)])

        @pl.when(wid == NW - 1)
        def _out_last():
            pltpu.sync_copy(acc.at[pl.ds(0, last_rows * D)],
                            out_hbm.at[pl.ds(lo * D, last_rows * D)])

    return acc_kernel


def kernel(node_features, nodepair, edge_attribute, node_neighbour_index,
           We1, be1, We2, be2, Wg1, bg1, Wg2, bg2):
    x = node_features
    N, D = x.shape
    E = edge_attribute.shape[0]
    NC, NS = _sc_geometry()
    NW = NC * NS
    NB = (-(-N // NW) + 7) // 8 * 8
    NBLK = E // _BE

    src = nodepair[0].astype(jnp.int32)
    dst = nodepair[1].astype(jnp.int32)
    nbr = node_neighbour_index.astype(jnp.int32)

    xs, xd = _make_gather(N, D, E, NC, NS)(x, src, dst)
    e_new = _edge_new_tc(xs, xd, edge_attribute,
                         We1, be1, We2, be2, Wg1, bg1, Wg2, bg2)

    nbr3 = nbr.reshape(NBLK, 1, _BE)
    hist = _hist_tc(nbr3, NB).reshape(NBLK, 32)
    blockpfx = jnp.cumsum(hist, axis=0) - hist
    totals = jnp.sum(hist, axis=0)
    bases = jnp.cumsum(totals) - totals
    bases3 = (bases[None, :] + blockpfx).reshape(NBLK, 1, 32)
    segb = jnp.zeros((48,), jnp.int32)
    segb = segb.at[:32].set(bases.astype(jnp.int32))
    segb = segb.at[32].set(jnp.int32(E))
    tril = jnp.tril(jnp.ones((_BE, _BE), jnp.float32), k=-1)
    p = _pos_tc(nbr3, bases3, tril, NB).reshape(E)

    srow, smeta = _make_permute(N, D, E, NC, NS)(e_new, nbr, p)
    out_flat = _make_accumulate(N, D, E, NC, NS)(srow, smeta, segb)
    return out_flat.reshape(N, D)


# speedup vs baseline: 1.2459x; 1.0037x over previous
import functools

import jax
import jax.numpy as jnp
from jax import lax
from jax.experimental import pallas as pl
from jax.experimental.pallas import tpu as pltpu
from jax.experimental.pallas import tpu_sc as plsc

_CH = 80
_BE = 512


def _sc_geometry():
    try:
        info = plsc.get_sparse_core_info()
        return info.num_cores, info.num_subcores
    except Exception:
        return 2, 16


def _make_gather(N, D, E, NC, NS):
    NW = NC * NS
    eper = E // NW
    nch = eper // _CH
    mesh = plsc.VectorSubcoreMesh(core_axis_name="c", subcore_axis_name="s")

    @functools.partial(
        pl.kernel,
        mesh=mesh,
        out_type=(
            jax.ShapeDtypeStruct((E, D), jnp.float32),
            jax.ShapeDtypeStruct((E, D), jnp.float32),
        ),
        scratch_types=[
            pltpu.VMEM((_CH,), jnp.int32),
            pltpu.VMEM((_CH,), jnp.int32),
            pltpu.VMEM((_CH, D), jnp.float32),
            pltpu.VMEM((_CH, D), jnp.float32),
            pltpu.SemaphoreType.DMA,
            pltpu.SemaphoreType.DMA,
        ],
    )
    def gather_kernel(x_hbm, src_hbm, dst_hbm, xs_hbm, xd_hbm,
                      sidx, didx, sbuf, dbuf, ssem, dsem):
        wid = lax.axis_index("s") * NC + lax.axis_index("c")
        ebase = wid * eper

        def body(j, carry):
            off = j * _CH
            pltpu.sync_copy(src_hbm.at[pl.ds(ebase + off, _CH)], sidx)
            pltpu.sync_copy(dst_hbm.at[pl.ds(ebase + off, _CH)], didx)
            cs = pltpu.async_copy(x_hbm.at[sidx], sbuf, ssem)
            cd = pltpu.async_copy(x_hbm.at[didx], dbuf, dsem)
            cs.wait()
            pltpu.sync_copy(sbuf, xs_hbm.at[pl.ds(ebase + off, _CH)])
            cd.wait()
            pltpu.sync_copy(dbuf, xd_hbm.at[pl.ds(ebase + off, _CH)])
            return carry

        lax.fori_loop(0, nch, body, 0)

    return gather_kernel


def _edge_mlp_block(xs_ref, xd_ref, ea_ref, We1_r, be1_r, We2_r, be2_r,
                    Wa_r, Wb_r, We_r, bg1_r, Wg2_r, bg2_r, out_ref):
    f32 = jnp.float32
    h1 = jnp.maximum(
        jnp.dot(ea_ref[:], We1_r[:], preferred_element_type=f32) + be1_r[:], 0.0)
    e = jnp.dot(h1, We2_r[:], preferred_element_type=f32) + be2_r[:]
    pre = (jnp.dot(xs_ref[:], Wa_r[:], preferred_element_type=f32)
           + jnp.dot(xd_ref[:], Wb_r[:], preferred_element_type=f32)
           + jnp.dot(e, We_r[:], preferred_element_type=f32)
           + bg1_r[:])
    out_ref[:] = (jnp.dot(jnp.maximum(pre, 0.0), Wg2_r[:],
                          preferred_element_type=f32) + bg2_r[:])


def _edge_new_tc(xs, xd, ea, We1, be1, We2, be2, Wg1, bg1, Wg2, bg2):
    E, D = xs.shape
    DE = ea.shape[1]
    HE = We1.shape[1]
    OE = We2.shape[1]
    HG = Wg1.shape[1]
    OG = Wg2.shape[1]
    BE = 2000
    grid = (E // BE,)
    Wa = Wg1[:D]
    Wb = Wg1[D:2 * D]
    We = Wg1[2 * D:]

    def full(shape):
        return pl.BlockSpec(shape, lambda i: (0,) * len(shape))

    return pl.pallas_call(
        _edge_mlp_block,
        grid=grid,
        in_specs=[
            pl.BlockSpec((BE, D), lambda i: (i, 0)),
            pl.BlockSpec((BE, D), lambda i: (i, 0)),
            pl.BlockSpec((BE, DE), lambda i: (i, 0)),
            full((DE, HE)), full((1, HE)),
            full((HE, OE)), full((1, OE)),
            full((D, HG)), full((D, HG)), full((OE, HG)), full((1, HG)),
            full((HG, OG)), full((1, OG)),
        ],
        out_specs=pl.BlockSpec((BE, OG), lambda i: (i, 0)),
        out_shape=jax.ShapeDtypeStruct((E, OG), jnp.float32),
    )(xs, xd, ea, We1, be1.reshape(1, -1), We2, be2.reshape(1, -1),
      Wa, Wb, We, bg1.reshape(1, -1), Wg2, bg2.reshape(1, -1))


def _hist_block_body(NB):
    def body(nbr_ref, out_ref):
        o = nbr_ref[0, 0] // NB
        iota = lax.broadcasted_iota(jnp.int32, (_BE, 32), 1)
        oh = (o[:, None] == iota).astype(jnp.float32)
        out_ref[0, 0] = jnp.sum(oh, axis=0)
    return body


def _hist_tc(nbr3, NB):
    NBLK = nbr3.shape[0]
    return pl.pallas_call(
        _hist_block_body(NB),
        grid=(NBLK,),
        in_specs=[pl.BlockSpec((1, 1, _BE), lambda i: (i, 0, 0))],
        out_specs=pl.BlockSpec((1, 1, 32), lambda i: (i, 0, 0)),
        out_shape=jax.ShapeDtypeStruct((NBLK, 1, 32), jnp.float32),
    )(nbr3)


def _pos_block_body(NB):
    def body(nbr_ref, bases_ref, tril_ref, out_ref):
        o = nbr_ref[0, 0] // NB
        iota = lax.broadcasted_iota(jnp.int32, (_BE, 32), 1)
        oh = (o[:, None] == iota).astype(jnp.float32)
        rank = jnp.dot(tril_ref[:], oh, preferred_element_type=jnp.float32)
        base = jnp.sum(bases_ref[0] * oh, axis=1)
        p = base + jnp.sum(rank * oh, axis=1)
        out_ref[0, 0] = p.astype(jnp.int32)
    return body


def _pos_tc(nbr3, bases3, tril, NB):
    NBLK = nbr3.shape[0]
    return pl.pallas_call(
        _pos_block_body(NB),
        grid=(NBLK,),
        in_specs=[
            pl.BlockSpec((1, 1, _BE), lambda i: (i, 0, 0)),
            pl.BlockSpec((1, 1, 32), lambda i: (i, 0, 0)),
            pl.BlockSpec((_BE, _BE), lambda i: (0, 0)),
        ],
        out_specs=pl.BlockSpec((1, 1, _BE), lambda i: (i, 0, 0)),
        out_shape=jax.ShapeDtypeStruct((NBLK, 1, _BE), jnp.int32),
    )(nbr3, bases3, tril)


def _make_permute(N, D, E, NC, NS):
    NW = NC * NS
    eper = E // NW
    nch = eper // _CH
    EP = E + _CH
    mesh = plsc.VectorSubcoreMesh(core_axis_name="c", subcore_axis_name="s")

    @functools.partial(
        pl.kernel,
        mesh=mesh,
        out_type=(
            jax.ShapeDtypeStruct((EP, D), jnp.float32),
            jax.ShapeDtypeStruct((EP, 16), jnp.int32),
        ),
        scratch_types=[
            pltpu.VMEM((_CH, D), jnp.float32),
            pltpu.VMEM((_CH,), jnp.int32),
            pltpu.VMEM((_CH,), jnp.int32),
            pltpu.VMEM((16, 16), jnp.int32),
            pltpu.SemaphoreType.DMA,
            pltpu.SemaphoreType.DMA,
        ],
    )
    def permute_kernel(e_hbm, nbr_hbm, p_hbm, srow_hbm, smeta_hbm,
                       ebuf, nbrbuf, pbuf, mstage, rsem, msem):
        wid = lax.axis_index("s") * NC + lax.axis_index("c")
        ebase = wid * eper
        iota16 = lax.iota(jnp.int32, 16)
        l0 = 1 - jnp.minimum(iota16, 1)
        l1 = jnp.maximum(0, 1 - jnp.abs(iota16 - 1))

        def chunk(j, carry):
            off = j * _CH
            pltpu.sync_copy(e_hbm.at[pl.ds(ebase + off, _CH)], ebuf)
            pltpu.sync_copy(nbr_hbm.at[pl.ds(ebase + off, _CH)], nbrbuf)
            pltpu.sync_copy(p_hbm.at[pl.ds(ebase + off, _CH)], pbuf)

            def group(g, carry2):
                pv = pbuf[pl.ds(g * 16, 16)]
                nv = nbrbuf[pl.ds(g * 16, 16)]
                copies = []
                for lane in range(16):
                    k = g * 16 + lane
                    eid = ebase + off + k
                    meta = eid * l0 + nv[lane] * l1
                    mstage[lane, :] = meta
                    pk = pv[lane]
                    copies.append(pltpu.async_copy(
                        ebuf.at[k], srow_hbm.at[pk], rsem))
                    copies.append(pltpu.async_copy(
                        mstage.at[lane], smeta_hbm.at[pk], msem))
                for c in copies:
                    c.wait()
                return carry2

            lax.fori_loop(0, _CH // 16, group, 0)
            return carry

        lax.fori_loop(0, nch, chunk, 0)

    return permute_kernel


def _make_accumulate(N, D, E, NC, NS):
    NW = NC * NS
    NB = (-(-N // NW) + 7) // 8 * 8
    DW = D // 16
    last_rows = N - (NW - 1) * NB
    EP = E + _CH
    mesh = plsc.VectorSubcoreMesh(core_axis_name="c", subcore_axis_name="s")

    @functools.partial(
        pl.kernel,
        mesh=mesh,
        out_type=jax.ShapeDtypeStruct((N * D,), jnp.float32),
        scratch_types=[
            pltpu.VMEM((_CH, D), jnp.float32),
            pltpu.VMEM((_CH, 16), jnp.int32),
            pltpu.VMEM((48,), jnp.int32),
            pltpu.VMEM(((NB + 8) * D,), jnp.float32),
            pltpu.VMEM(((NB + 8) * 16,), jnp.float32),
        ],
    )
    def acc_kernel(srow_hbm, smeta_hbm, segb_hbm, out_hbm,
                   srbuf, smbuf, segbuf, acc, cnt):
        cid = lax.axis_index("c")
        sid = lax.axis_index("s")
        wid = sid * NC + cid
        lo = wid * NB
        zf16 = jnp.zeros((16,), jnp.float32)
        one16 = jnp.ones((16,), jnp.float32)

        def zacc(i, c):
            acc[pl.ds(i * 16, 16)] = zf16
            return c

        lax.fori_loop(0, (NB + 8) * D // 16, zacc, 0)

        def zcnt(i, c):
            cnt[pl.ds(i * 16, 16)] = zf16
            return c

        lax.fori_loop(0, NB + 8, zcnt, 0)

        pltpu.sync_copy(segb_hbm, segbuf)
        sv0 = segbuf[pl.ds(0, 16)]
        sv1 = segbuf[pl.ds(16, 16)]
        sv2 = segbuf[pl.ds(32, 16)]
        b_lo = jnp.int32(0)
        b_hi = jnp.int32(0)
        for t in range(33):
            val = (sv0, sv1, sv2)[t // 16][t % 16]
            b_lo = jnp.where(wid == t, val, b_lo)
            b_hi = jnp.where(wid + 1 == t, val, b_hi)

        start = pl.multiple_of(b_lo & jnp.int32(-8), 8)
        nchunks = (b_hi - start + _CH - 1) // _CH

        def chunk(j, carry):
            g0 = pl.multiple_of(start + j * _CH, 8)
            pltpu.sync_copy(srow_hbm.at[pl.ds(g0, _CH)], srbuf)
            pltpu.sync_copy(smeta_hbm.at[pl.ds(g0, _CH)], smbuf)

            def edge_group(g, c2):
                kbase = g * 16
                for lane in range(16):
                    k = kbase + lane
                    mrow = smbuf[k, pl.ds(0, 16)]
                    r0 = mrow[1] - lo
                    kg = g0 + k
                    r1 = jnp.where(kg >= b_hi, NB, r0)
                    r1 = jnp.where(r0 < 0, NB, r1)
                    r1 = jnp.where(r0 >= NB, NB, r1)
                    base = r1 * D
                    for j8 in range(DW):
                        acc[pl.ds(base + j8 * 16, 16)] = (
                            acc[pl.ds(base + j8 * 16, 16)]
                            + srbuf[k, pl.ds(j8 * 16, 16)])
                    cb = r1 * 16
                    cnt[pl.ds(cb, 16)] = cnt[pl.ds(cb, 16)] + one16
                return c2

            lax.fori_loop(0, _CH // 16, edge_group, 0)
            return carry

        lax.fori_loop(0, nchunks, chunk, 0)

        def div_row(r, c):
            cvec = jnp.maximum(cnt[pl.ds(r * 16, 16)], 1.0)
            for j8 in range(DW):
                acc[pl.ds(r * D + j8 * 16, 16)] = (
                    acc[pl.ds(r * D + j8 * 16, 16)] / cvec)
            return c

        lax.fori_loop(0, NB, div_row, 0)

        @pl.when(wid < NW - 1)
        def _out_full():
            pltpu.sync_copy(acc.at[pl.ds(0, NB * D)],
                            out_hbm.at[pl.ds(lo * D, NB * D)])

        @pl.when(wid == NW - 1)
        def _out_last():
            pltpu.sync_copy(acc.at[pl.ds(0, last_rows * D)],
                            out_hbm.at[pl.ds(lo * D, last_rows * D)])

    return acc_kernel


def kernel(node_features, nodepair, edge_attribute, node_neighbour_index,
           We1, be1, We2, be2, Wg1, bg1, Wg2, bg2):
    x = node_features
    N, D = x.shape
    E = edge_attribute.shape[0]
    NC, NS = _sc_geometry()
    NW = NC * NS
    NB = (-(-N // NW) + 7) // 8 * 8
    NBLK = E // _BE

    src = nodepair[0].astype(jnp.int32)
    dst = nodepair[1].astype(jnp.int32)
    nbr = node_neighbour_index.astype(jnp.int32)

    xs, xd = _make_gather(N, D, E, NC, NS)(x, src, dst)
    e_new = _edge_new_tc(xs, xd, edge_attribute,
                         We1, be1, We2, be2, Wg1, bg1, Wg2, bg2)

    nbr3 = nbr.reshape(NBLK, 1, _BE)
    hist = _hist_tc(nbr3, NB).reshape(NBLK, 32)
    blockpfx = jnp.cumsum(hist, axis=0) - hist
    totals = jnp.sum(hist, axis=0)
    bases = jnp.cumsum(totals) - totals
    bases3 = (bases[None, :] + blockpfx).reshape(NBLK, 1, 32)
    segb = jnp.zeros((48,), jnp.int32)
    segb = segb.at[:32].set(bases.astype(jnp.int32))
    segb = segb.at[32].set(jnp.int32(E))
    tril = jnp.tril(jnp.ones((_BE, _BE), jnp.float32), k=-1)
    p = _pos_tc(nbr3, bases3, tril, NB).reshape(E)

    srow, smeta = _make_permute(N, D, E, NC, NS)(e_new, nbr, p)
    out_flat = _make_accumulate(N, D, E, NC, NS)(srow, smeta, segb)
    return out_flat.reshape(N, D)
